# baseline (speedup 1.0000x reference)
import jax
import jax.numpy as jnp
from jax import lax
from jax.experimental import pallas as pl
from jax.experimental.pallas import tpu as pltpu
from jax.experimental.pallas import tpu_sc as plsc

N = 10000
E = 160000
G = 128
D_IN = 48
D_EDGE = 4
HID = 128

NC = 2
NS = 16
NW = NC * NS
L = 16

C = 64

E2 = E + N
E2P = 172032
EPW = E2P // NW
NCH = EPW // C

EAP = 163840
EPWA = EAP // NW
NCHA = EPWA // C

NACC = 10240
RPT = NACC // NS
DENR = 128
DPT = DENR // NS
NACCT = NACC + DENR
NDR = NACC // HID

f32 = jnp.float32


def _lanesum(v):
    parts = [v[i] for i in range(L)]
    while len(parts) > 1:
        parts = [parts[i] + parts[i + 1] for i in range(0, len(parts), 2)]
    return parts[0]


def _zero_rows(buf, nrows):
    def body(i, _):
        for j in range(HID // L):
            buf[i, pl.ds(j * L, L)] = jnp.zeros((L,), f32)
        return 0
    lax.fori_loop(0, nrows, body, 0)


def _edge_body(xl_hbm, xr_hbm, src_hbm, dst_hbm, dstx_hbm, eaw_hbm,
               att_hbm, out_hbm, acc_sh, src_v, dst_v, dstx_v, didx_v,
               xlg, xrg, eawg, mbuf, dbuf, dsmall, att_v, gsem):
    cid = lax.axis_index("c")
    sid = lax.axis_index("s")
    wid = sid * NC + cid
    r0 = sid * RPT
    d0 = NACC + sid * DPT

    _zero_rows(mbuf, C)
    _zero_rows(dsmall, DPT)
    for k in range(RPT // C):
        pltpu.sync_copy(mbuf, acc_sh.at[pl.ds(r0 + k * C, C)])
    pltpu.sync_copy(dsmall, acc_sh.at[pl.ds(d0, DPT)])
    pltpu.sync_copy(att_hbm, att_v)
    plsc.subcore_barrier()

    base_e = wid * EPW
    ii = lax.iota(jnp.int32, L)

    def chunk(c, _):
        e0 = base_e + c * C
        pltpu.sync_copy(src_hbm.at[pl.ds(e0, C)], src_v)
        pltpu.sync_copy(dst_hbm.at[pl.ds(e0, C)], dst_v)
        pltpu.sync_copy(dstx_hbm.at[pl.ds(e0, C + L)], dstx_v)
        pltpu.sync_copy(eaw_hbm.at[pl.ds(e0, C)], eawg)
        cp1 = pltpu.async_copy(xl_hbm.at[src_v], xlg, gsem)
        cp2 = pltpu.async_copy(xr_hbm.at[dst_v], xrg, gsem)

        def didx(g, _):
            dv = dstx_v[pl.ds(g * L, L)]
            didx_v[pl.ds(g * L, L)] = NACC + lax.shift_right_logical(dv, 7)
            return 0

        lax.fori_loop(0, C // L, didx, 0)
        cp1.wait()
        cp2.wait()

        @plsc.parallel_loop(0, C, unroll=4)
        def edge(i):
            part = jnp.zeros((L,), f32)
            for j in range(HID // L):
                h = (xlg[i, pl.ds(j * L, L)] + xrg[i, pl.ds(j * L, L)]
                     + eawg[i, pl.ds(j * L, L)])
                h = jnp.where(h > 0, h, 0.2 * h)
                part = part + h * att_v[pl.ds(j * L, L)]
            lsum = _lanesum(part)
            lsum = jnp.minimum(jnp.maximum(lsum, -50.0), 50.0)
            evec = jnp.exp(jnp.zeros((L,), f32) + lsum)
            evec = jnp.where(e0 + i < E2, evec, jnp.zeros((L,), f32))
            for j in range(HID // L):
                mbuf[i, pl.ds(j * L, L)] = evec * xlg[i, pl.ds(j * L, L)]
            dmod = lax.bitwise_and(dstx_v[pl.ds(i, L)][0], HID - 1)
            for j in range(HID // L):
                dbuf[i, pl.ds(j * L, L)] = jnp.where(
                    ii + (j * L) == dmod, evec, jnp.zeros((L,), f32))

        pltpu.sync_copy(mbuf, acc_sh.at[dst_v], add=True)
        pltpu.sync_copy(dbuf, acc_sh.at[didx_v], add=True)
        return 0

    lax.fori_loop(0, NCH, chunk, 0)
    plsc.subcore_barrier()
    for k in range(RPT // C):
        pltpu.sync_copy(acc_sh.at[pl.ds(r0 + k * C, C)], mbuf)
        pltpu.sync_copy(mbuf, out_hbm.at[cid, pl.ds(r0 + k * C, C)])
    pltpu.sync_copy(acc_sh.at[pl.ds(d0, DPT)], dsmall)
    pltpu.sync_copy(dsmall, out_hbm.at[cid, pl.ds(d0, DPT)])


def _sc_edge(xl, xr, src2, dst2, eaw, att):
    mesh = plsc.VectorSubcoreMesh(core_axis_name="c", subcore_axis_name="s")
    return pl.kernel(
        _edge_body,
        out_type=jax.ShapeDtypeStruct((NC, NACCT, HID), f32),
        mesh=mesh,
        scratch_types=[
            pltpu.VMEM_SHARED((NACCT, HID), f32),
            pltpu.VMEM((C,), jnp.int32),
            pltpu.VMEM((C,), jnp.int32),
            pltpu.VMEM((C + L,), jnp.int32),
            pltpu.VMEM((C,), jnp.int32),
            pltpu.VMEM((C, HID), f32),
            pltpu.VMEM((C, HID), f32),
            pltpu.VMEM((C, HID), f32),
            pltpu.VMEM((C, HID), f32),
            pltpu.VMEM((C, HID), f32),
            pltpu.VMEM((DPT, HID), f32),
            pltpu.VMEM((HID,), f32),
            pltpu.SemaphoreType.DMA,
        ],
        name="gat_edge_sc",
    )(xl, xr, src2, dst2, dst2, eaw, att)


EAWB = 8192


def _eaw_body(ea_ref, we_ref, out_ref):
    out_ref[...] = jnp.dot(ea_ref[...], we_ref[...],
                           preferred_element_type=f32)


def _tc_eaw(ea2, we):
    return pl.pallas_call(
        _eaw_body,
        grid=(E2P // EAWB,),
        in_specs=[
            pl.BlockSpec((EAWB, D_EDGE), lambda i: (i, 0)),
            pl.BlockSpec((D_EDGE, HID), lambda i: (0, 0)),
        ],
        out_specs=pl.BlockSpec((EAWB, HID), lambda i: (i, 0)),
        out_shape=jax.ShapeDtypeStruct((E2P, HID), f32),
    )(ea2, we)


def _loopea_body(dst_hbm, ea_hbm, out_hbm, acc_sh, dst_v, ea_v, vbuf):
    cid = lax.axis_index("c")
    sid = lax.axis_index("s")
    wid = sid * NC + cid
    r0 = sid * RPT

    _zero_rows(vbuf, C)
    for k in range(RPT // C):
        pltpu.sync_copy(vbuf, acc_sh.at[pl.ds(r0 + k * C, C)])
    plsc.subcore_barrier()

    base_e = wid * EPWA
    ii = lax.iota(jnp.int32, L)

    def chunk(c, _):
        e0 = base_e + c * C
        pltpu.sync_copy(dst_hbm.at[pl.ds(e0, C)], dst_v)
        pltpu.sync_copy(ea_hbm.at[pl.ds(e0 * D_EDGE, C * D_EDGE + L)], ea_v)

        @plsc.parallel_loop(0, C, unroll=4)
        def edge(i):
            sl = ea_v[pl.ds(D_EDGE * i, L)]
            mval = jnp.where(e0 + i < E, 1.0, 0.0).astype(f32)
            row = jnp.where(ii < D_EDGE, sl, 0.0)
            row = row + jnp.where(ii == D_EDGE, mval, 0.0)
            vbuf[i, pl.ds(0, L)] = row

        pltpu.sync_copy(vbuf, acc_sh.at[dst_v], add=True)
        return 0

    lax.fori_loop(0, NCHA, chunk, 0)
    plsc.subcore_barrier()
    for k in range(RPT // C):
        pltpu.sync_copy(acc_sh.at[pl.ds(r0 + k * C, C)], vbuf)
        pltpu.sync_copy(vbuf, out_hbm.at[cid, pl.ds(r0 + k * C, C)])


def _sc_loopea(dstA, eaA):
    mesh = plsc.VectorSubcoreMesh(core_axis_name="c", subcore_axis_name="s")
    return pl.kernel(
        _loopea_body,
        out_type=jax.ShapeDtypeStruct((NC, NACC, HID), f32),
        mesh=mesh,
        scratch_types=[
            pltpu.VMEM_SHARED((NACC, HID), f32),
            pltpu.VMEM((C,), jnp.int32),
            pltpu.VMEM((C * D_EDGE + L,), f32),
            pltpu.VMEM((C, HID), f32),
        ],
        name="loopea_sc",
    )(dstA, eaA)


def _k0_body(x_ref, wl_ref, wr_ref, accA_ref, xl_out, xr_out, lea_out):
    x = x_ref[...]
    xl_out[...] = jnp.dot(x, wl_ref[...], preferred_element_type=f32)
    xr_out[...] = jnp.dot(x, wr_ref[...], preferred_element_type=f32)
    a = accA_ref[0, :N] + accA_ref[1, :N]
    lea_out[...] = a[:, 0:D_EDGE] / jnp.maximum(a[:, D_EDGE:D_EDGE + 1], 1.0)


def _tc_k0(x, wl, wr, accA):
    return pl.pallas_call(
        _k0_body,
        out_shape=[
            jax.ShapeDtypeStruct((N, HID), f32),
            jax.ShapeDtypeStruct((N, HID), f32),
            jax.ShapeDtypeStruct((N, D_EDGE), f32),
        ],
    )(x, wl, wr, accA)


def _nodes_h(acc_ref, b_ref):
    nump = acc_ref[0, :NACC] + acc_ref[1, :NACC]
    dd = acc_ref[0, NACC:NACC + NDR] + acc_ref[1, NACC:NACC + NDR]
    rd = 1.0 / (dd + 1e-16)
    num3 = jnp.reshape(nump, (NDR, HID, HID))
    d3 = jnp.eye(HID, dtype=f32)[None] * rd[:, None, :]
    h3 = lax.dot_general(d3, num3, (((2,), (1,)), ((0,), (0,))),
                         preferred_element_type=f32)
    return jnp.reshape(h3, (NACC, HID))[:N] + b_ref[...]


def _comb_body(acc_ref, b_ref, w_ref, bias_ref, ms_ref, a_ref, wl_ref, wr_ref,
               xl_out, xr_out):
    h = _nodes_h(acc_ref, b_ref)
    mean = jnp.mean(h, axis=0, keepdims=True)
    out = h - ms_ref[...] * mean
    var = jnp.mean(out * out, axis=0, keepdims=True)
    hn = w_ref[...] * out * lax.rsqrt(var + 1e-5) + bias_ref[...]
    ap = a_ref[0, 0]
    hp = jnp.where(hn >= 0, hn, ap * hn)
    xl_out[...] = jnp.dot(hp, wl_ref[...], preferred_element_type=f32)
    xr_out[...] = jnp.dot(hp, wr_ref[...], preferred_element_type=f32)


def _tc_comb(acc2, b, norm, a, wl, wr):
    return pl.pallas_call(
        _comb_body,
        out_shape=[
            jax.ShapeDtypeStruct((N, HID), f32),
            jax.ShapeDtypeStruct((N, HID), f32),
        ],
    )(acc2, b.reshape(1, HID), norm["weight"].reshape(1, HID),
      norm["bias"].reshape(1, HID), norm["mean_scale"].reshape(1, HID),
      a.reshape(1, 1), wl, wr)


def _ro_body(acc_ref, b_ref, batch_ref, wri, bri, wro, bro, wfn, bfn, wf2, bf2,
             out_out, emb_out):
    h = _nodes_h(acc_ref, b_ref)
    ip = jnp.dot(h, wri[...], preferred_element_type=f32) + bri[...]
    ip = jnp.where(ip > 0, ip, 0.01 * ip)
    op = jnp.dot(ip, wro[...], preferred_element_type=f32) + bro[...]
    gids = lax.broadcasted_iota(jnp.int32, (G, N), 0)
    oh = (gids == batch_ref[...]).astype(f32)
    gsum = jnp.dot(oh, op, preferred_element_type=f32)
    gcnt = jnp.sum(oh, axis=1, keepdims=True)
    p1 = gsum / jnp.maximum(gcnt, 1.0)
    z = jnp.dot(p1, wfn[...], preferred_element_type=f32) + bfn[...]
    emb = jnp.where(z > 0, z, 0.01 * z)
    out_out[...] = jnp.dot(emb, wf2[...], preferred_element_type=f32) + bf2[...]
    emb_out[...] = emb


def _tc_readout(acc2, b, batch, p_ri, p_ro, p_fn, p_f2):
    return pl.pallas_call(
        _ro_body,
        out_shape=[
            jax.ShapeDtypeStruct((G, 3), f32),
            jax.ShapeDtypeStruct((G, 64), f32),
        ],
    )(acc2, b.reshape(1, HID), batch.reshape(1, N),
      p_ri["W"].T, p_ri["b"].reshape(1, 64),
      p_ro["W"].T, p_ro["b"].reshape(1, 64),
      p_fn["W"].T, p_fn["b"].reshape(1, 64),
      p_f2["W"].T, p_f2["b"].reshape(1, 3))


def kernel(x, edge_index, edge_attr, batch, params):
    src, dst = edge_index[0], edge_index[1]
    idt = src.dtype
    ar = jnp.arange(N, dtype=idt)

    dstA = jnp.concatenate([dst, jnp.zeros((EAP - E,), idt)])
    eaA = jnp.concatenate(
        [edge_attr.reshape(-1), jnp.zeros(((EAP - E) * D_EDGE + L,), f32)])
    accA = _sc_loopea(dstA, eaA)

    convs = params["convs"]
    xl, xr, lea = _tc_k0(x, convs[0]["Wl"], convs[0]["Wr"], accA)

    src2 = jnp.concatenate([src, ar, jnp.zeros((E2P - E2,), idt)])
    dst2 = jnp.concatenate([dst, ar, jnp.zeros((E2P - E2 + L,), idt)])
    ea2 = jnp.concatenate(
        [edge_attr, lea, jnp.zeros((E2P - E2, D_EDGE), f32)])

    out = emb = None
    for i in range(5):
        p = convs[i]
        eaw = _tc_eaw(ea2, p["We"])
        acc2 = _sc_edge(xl, xr, src2, dst2, eaw, p["att"])
        if i < 4:
            xl, xr = _tc_comb(acc2, p["b"], params["norms"][i],
                              params["prelu"], convs[i + 1]["Wl"],
                              convs[i + 1]["Wr"])
        else:
            out, emb = _tc_readout(acc2, p["b"], batch,
                                   params["readout_in"], params["readout_out"],
                                   params["fc_n"], params["fc_n2"])
    return (out, emb)


# speedup vs baseline: 1.1432x; 1.1432x over previous
import jax
import jax.numpy as jnp
from jax import lax
from jax.experimental import pallas as pl
from jax.experimental.pallas import tpu as pltpu
from jax.experimental.pallas import tpu_sc as plsc

N = 10000
E = 160000
G = 128
D_IN = 48
D_EDGE = 4
HID = 128

NC = 2
NS = 16
NW = NC * NS
L = 16

C = 64

E2 = E + N
E2P = 172032
EPW = E2P // NW
NCH = EPW // C

EAP = 163840
EPWA = EAP // NW
NCHA = EPWA // C

NACC = 10240
RPT = NACC // NS
DENR = 128
DPT = DENR // NS
NACCT = NACC + DENR
NDR = NACC // HID

f32 = jnp.float32


def _lanesum(v):
    parts = [v[i] for i in range(L)]
    while len(parts) > 1:
        parts = [parts[i] + parts[i + 1] for i in range(0, len(parts), 2)]
    return parts[0]


def _zero_rows(buf, nrows):
    def body(i, _):
        for j in range(HID // L):
            buf[i, pl.ds(j * L, L)] = jnp.zeros((L,), f32)
        return 0
    lax.fori_loop(0, nrows, body, 0)


def _edge_body(xl_hbm, xr_hbm, src_hbm, dst_hbm, dstx_hbm, eaw_hbm,
               att_hbm, out_hbm, acc_sh, src_v, dst_v, dstx_v, didx_v,
               xlg, xrg, eawg, mbuf, dbuf, dsmall, att_v, gsem):
    cid = lax.axis_index("c")
    sid = lax.axis_index("s")
    wid = sid * NC + cid
    r0 = sid * RPT
    d0 = NACC + sid * DPT

    _zero_rows(mbuf, C)
    _zero_rows(dsmall, DPT)
    for k in range(RPT // C):
        pltpu.sync_copy(mbuf, acc_sh.at[pl.ds(r0 + k * C, C)])
    pltpu.sync_copy(dsmall, acc_sh.at[pl.ds(d0, DPT)])
    pltpu.sync_copy(att_hbm, att_v)
    plsc.subcore_barrier()

    base_e = wid * EPW
    ii = lax.iota(jnp.int32, L)

    def chunk(c, _):
        e0 = base_e + c * C
        pltpu.sync_copy(src_hbm.at[pl.ds(e0, C)], src_v)
        pltpu.sync_copy(dst_hbm.at[pl.ds(e0, C)], dst_v)
        cp1 = pltpu.async_copy(xl_hbm.at[src_v], xlg, gsem)
        cp2 = pltpu.async_copy(xr_hbm.at[dst_v], xrg, gsem)
        pltpu.sync_copy(dstx_hbm.at[pl.ds(e0, C + L)], dstx_v)
        pltpu.sync_copy(eaw_hbm.at[pl.ds(e0, C)], eawg)

        def didx(g, _):
            dv = dstx_v[pl.ds(g * L, L)]
            didx_v[pl.ds(g * L, L)] = NACC + lax.shift_right_logical(dv, 7)
            return 0

        lax.fori_loop(0, C // L, didx, 0)
        cp1.wait()
        cp2.wait()

        @plsc.parallel_loop(0, C, unroll=4)
        def edge(i):
            part = jnp.zeros((L,), f32)
            for j in range(HID // L):
                h = (xlg[i, pl.ds(j * L, L)] + xrg[i, pl.ds(j * L, L)]
                     + eawg[i, pl.ds(j * L, L)])
                h = jnp.where(h > 0, h, 0.2 * h)
                part = part + h * att_v[pl.ds(j * L, L)]
            lsum = _lanesum(part)
            lsum = jnp.minimum(jnp.maximum(lsum, -50.0), 50.0)
            evec = jnp.exp(jnp.zeros((L,), f32) + lsum)
            evec = jnp.where(e0 + i < E2, evec, jnp.zeros((L,), f32))
            for j in range(HID // L):
                mbuf[i, pl.ds(j * L, L)] = evec * xlg[i, pl.ds(j * L, L)]
            dmod = lax.bitwise_and(dstx_v[pl.ds(i, L)][0], HID - 1)
            for j in range(HID // L):
                dbuf[i, pl.ds(j * L, L)] = jnp.where(
                    ii + (j * L) == dmod, evec, jnp.zeros((L,), f32))

        pltpu.sync_copy(mbuf, acc_sh.at[dst_v], add=True)
        pltpu.sync_copy(dbuf, acc_sh.at[didx_v], add=True)
        return 0

    lax.fori_loop(0, NCH, chunk, 0)
    plsc.subcore_barrier()
    for k in range(RPT // C):
        pltpu.sync_copy(acc_sh.at[pl.ds(r0 + k * C, C)], mbuf)
        pltpu.sync_copy(mbuf, out_hbm.at[cid, pl.ds(r0 + k * C, C)])
    pltpu.sync_copy(acc_sh.at[pl.ds(d0, DPT)], dsmall)
    pltpu.sync_copy(dsmall, out_hbm.at[cid, pl.ds(d0, DPT)])


def _sc_edge(xl, xr, src2, dst2, eaw, att):
    mesh = plsc.VectorSubcoreMesh(core_axis_name="c", subcore_axis_name="s")
    return pl.kernel(
        _edge_body,
        out_type=jax.ShapeDtypeStruct((NC, NACCT, HID), f32),
        mesh=mesh,
        scratch_types=[
            pltpu.VMEM_SHARED((NACCT, HID), f32),
            pltpu.VMEM((C,), jnp.int32),
            pltpu.VMEM((C,), jnp.int32),
            pltpu.VMEM((C + L,), jnp.int32),
            pltpu.VMEM((C,), jnp.int32),
            pltpu.VMEM((C, HID), f32),
            pltpu.VMEM((C, HID), f32),
            pltpu.VMEM((C, HID), f32),
            pltpu.VMEM((C, HID), f32),
            pltpu.VMEM((C, HID), f32),
            pltpu.VMEM((DPT, HID), f32),
            pltpu.VMEM((HID,), f32),
            pltpu.SemaphoreType.DMA,
        ],
        name="gat_edge_sc",
    )(xl, xr, src2, dst2, dst2, eaw, att)


EAWB = 8192


def _eaw_body(ea_ref, we_ref, out_ref):
    out_ref[...] = jnp.dot(ea_ref[...], we_ref[...],
                           preferred_element_type=f32)


def _tc_eaw(ea2, we):
    return pl.pallas_call(
        _eaw_body,
        grid=(E2P // EAWB,),
        in_specs=[
            pl.BlockSpec((EAWB, D_EDGE), lambda i: (i, 0)),
            pl.BlockSpec((D_EDGE, HID), lambda i: (0, 0)),
        ],
        out_specs=pl.BlockSpec((EAWB, HID), lambda i: (i, 0)),
        out_shape=jax.ShapeDtypeStruct((E2P, HID), f32),
    )(ea2, we)


def _loopea_body(dst_hbm, ea_hbm, out_hbm, acc_sh, dst_v, ea_v, vbuf):
    cid = lax.axis_index("c")
    sid = lax.axis_index("s")
    wid = sid * NC + cid
    r0 = sid * RPT

    _zero_rows(vbuf, C)
    for k in range(RPT // C):
        pltpu.sync_copy(vbuf, acc_sh.at[pl.ds(r0 + k * C, C)])
    plsc.subcore_barrier()

    base_e = wid * EPWA
    ii = lax.iota(jnp.int32, L)

    def chunk(c, _):
        e0 = base_e + c * C
        pltpu.sync_copy(dst_hbm.at[pl.ds(e0, C)], dst_v)
        pltpu.sync_copy(ea_hbm.at[pl.ds(e0 * D_EDGE, C * D_EDGE + L)], ea_v)

        @plsc.parallel_loop(0, C, unroll=4)
        def edge(i):
            sl = ea_v[pl.ds(D_EDGE * i, L)]
            mval = jnp.where(e0 + i < E, 1.0, 0.0).astype(f32)
            row = jnp.where(ii < D_EDGE, sl, 0.0)
            row = row + jnp.where(ii == D_EDGE, mval, 0.0)
            vbuf[i, pl.ds(0, L)] = row

        pltpu.sync_copy(vbuf, acc_sh.at[dst_v], add=True)
        return 0

    lax.fori_loop(0, NCHA, chunk, 0)
    plsc.subcore_barrier()
    for k in range(RPT // C):
        pltpu.sync_copy(acc_sh.at[pl.ds(r0 + k * C, C)], vbuf)
        pltpu.sync_copy(vbuf, out_hbm.at[cid, pl.ds(r0 + k * C, C)])


def _sc_loopea(dstA, eaA):
    mesh = plsc.VectorSubcoreMesh(core_axis_name="c", subcore_axis_name="s")
    return pl.kernel(
        _loopea_body,
        out_type=jax.ShapeDtypeStruct((NC, NACC, HID), f32),
        mesh=mesh,
        scratch_types=[
            pltpu.VMEM_SHARED((NACC, HID), f32),
            pltpu.VMEM((C,), jnp.int32),
            pltpu.VMEM((C * D_EDGE + L,), f32),
            pltpu.VMEM((C, HID), f32),
        ],
        name="loopea_sc",
    )(dstA, eaA)


def _k0_body(x_ref, wl_ref, wr_ref, accA_ref, xl_out, xr_out, lea_out):
    x = x_ref[...]
    xl_out[...] = jnp.dot(x, wl_ref[...], preferred_element_type=f32)
    xr_out[...] = jnp.dot(x, wr_ref[...], preferred_element_type=f32)
    a = accA_ref[0, :N] + accA_ref[1, :N]
    lea_out[...] = a[:, 0:D_EDGE] / jnp.maximum(a[:, D_EDGE:D_EDGE + 1], 1.0)


def _tc_k0(x, wl, wr, accA):
    return pl.pallas_call(
        _k0_body,
        out_shape=[
            jax.ShapeDtypeStruct((N, HID), f32),
            jax.ShapeDtypeStruct((N, HID), f32),
            jax.ShapeDtypeStruct((N, D_EDGE), f32),
        ],
    )(x, wl, wr, accA)


def _nodes_h(acc_ref, b_ref):
    nump = acc_ref[0, :NACC] + acc_ref[1, :NACC]
    dd = acc_ref[0, NACC:NACC + NDR] + acc_ref[1, NACC:NACC + NDR]
    rd = 1.0 / (dd + 1e-16)
    num3 = jnp.reshape(nump, (NDR, HID, HID))
    d3 = jnp.eye(HID, dtype=f32)[None] * rd[:, None, :]
    h3 = lax.dot_general(d3, num3, (((2,), (1,)), ((0,), (0,))),
                         preferred_element_type=f32)
    return jnp.reshape(h3, (NACC, HID))[:N] + b_ref[...]


def _comb_body(acc_ref, b_ref, w_ref, bias_ref, ms_ref, a_ref, wl_ref, wr_ref,
               xl_out, xr_out):
    h = _nodes_h(acc_ref, b_ref)
    mean = jnp.mean(h, axis=0, keepdims=True)
    out = h - ms_ref[...] * mean
    var = jnp.mean(out * out, axis=0, keepdims=True)
    hn = w_ref[...] * out * lax.rsqrt(var + 1e-5) + bias_ref[...]
    ap = a_ref[0, 0]
    hp = jnp.where(hn >= 0, hn, ap * hn)
    xl_out[...] = jnp.dot(hp, wl_ref[...], preferred_element_type=f32)
    xr_out[...] = jnp.dot(hp, wr_ref[...], preferred_element_type=f32)


def _tc_comb(acc2, b, norm, a, wl, wr):
    return pl.pallas_call(
        _comb_body,
        out_shape=[
            jax.ShapeDtypeStruct((N, HID), f32),
            jax.ShapeDtypeStruct((N, HID), f32),
        ],
    )(acc2, b.reshape(1, HID), norm["weight"].reshape(1, HID),
      norm["bias"].reshape(1, HID), norm["mean_scale"].reshape(1, HID),
      a.reshape(1, 1), wl, wr)


def _ro_body(acc_ref, b_ref, batch_ref, wri, bri, wro, bro, wfn, bfn, wf2, bf2,
             out_out, emb_out):
    h = _nodes_h(acc_ref, b_ref)
    ip = jnp.dot(h, wri[...], preferred_element_type=f32) + bri[...]
    ip = jnp.where(ip > 0, ip, 0.01 * ip)
    op = jnp.dot(ip, wro[...], preferred_element_type=f32) + bro[...]
    gids = lax.broadcasted_iota(jnp.int32, (G, N), 0)
    oh = (gids == batch_ref[...]).astype(f32)
    gsum = jnp.dot(oh, op, preferred_element_type=f32)
    gcnt = jnp.sum(oh, axis=1, keepdims=True)
    p1 = gsum / jnp.maximum(gcnt, 1.0)
    z = jnp.dot(p1, wfn[...], preferred_element_type=f32) + bfn[...]
    emb = jnp.where(z > 0, z, 0.01 * z)
    out_out[...] = jnp.dot(emb, wf2[...], preferred_element_type=f32) + bf2[...]
    emb_out[...] = emb


def _tc_readout(acc2, b, batch, p_ri, p_ro, p_fn, p_f2):
    return pl.pallas_call(
        _ro_body,
        out_shape=[
            jax.ShapeDtypeStruct((G, 3), f32),
            jax.ShapeDtypeStruct((G, 64), f32),
        ],
    )(acc2, b.reshape(1, HID), batch.reshape(1, N),
      p_ri["W"].T, p_ri["b"].reshape(1, 64),
      p_ro["W"].T, p_ro["b"].reshape(1, 64),
      p_fn["W"].T, p_fn["b"].reshape(1, 64),
      p_f2["W"].T, p_f2["b"].reshape(1, 3))


def kernel(x, edge_index, edge_attr, batch, params):
    src, dst = edge_index[0], edge_index[1]
    idt = src.dtype
    ar = jnp.arange(N, dtype=idt)

    dstA = jnp.concatenate([dst, jnp.zeros((EAP - E,), idt)])
    eaA = jnp.concatenate(
        [edge_attr.reshape(-1), jnp.zeros(((EAP - E) * D_EDGE + L,), f32)])
    accA = _sc_loopea(dstA, eaA)

    convs = params["convs"]
    xl, xr, lea = _tc_k0(x, convs[0]["Wl"], convs[0]["Wr"], accA)

    src2 = jnp.concatenate([src, ar, jnp.zeros((E2P - E2,), idt)])
    dst2 = jnp.concatenate([dst, ar, jnp.zeros((E2P - E2 + L,), idt)])
    ea2 = jnp.concatenate(
        [edge_attr, lea, jnp.zeros((E2P - E2, D_EDGE), f32)])

    out = emb = None
    for i in range(5):
        p = convs[i]
        eaw = _tc_eaw(ea2, p["We"])
        acc2 = _sc_edge(xl, xr, src2, dst2, eaw, p["att"])
        if i < 4:
            xl, xr = _tc_comb(acc2, p["b"], params["norms"][i],
                              params["prelu"], convs[i + 1]["Wl"],
                              convs[i + 1]["Wr"])
        else:
            out, emb = _tc_readout(acc2, p["b"], batch,
                                   params["readout_in"], params["readout_out"],
                                   params["fc_n"], params["fc_n2"])
    return (out, emb)
